# baseline (speedup 1.0000x reference)
import functools

import jax
import jax.numpy as jnp
from jax import lax
from jax.experimental import pallas as pl
from jax.experimental.pallas import tpu as pltpu
from jax.experimental.pallas import tpu_sc as plsc

_NUM_ROWS = 1000000
_DIM = 64
_CHUNK = 800
_NCHUNKS = _NUM_ROWS // _CHUNK
_NW = 32
_MAX_PER_W = -(-_NCHUNKS // _NW)


def _sc_copy(src_hbm, out_hbm, buf):
    c = lax.axis_index("c")
    s = lax.axis_index("s")
    wid = s * 2 + c
    for i in range(_MAX_PER_W):
        ci = i * _NW + wid

        @pl.when(ci < _NCHUNKS)
        def _():
            base = ci * _CHUNK
            pltpu.sync_copy(src_hbm.at[pl.ds(base, _CHUNK)], buf)
            pltpu.sync_copy(buf, out_hbm.at[pl.ds(base, _CHUNK)])


def kernel(embeddings):
    mesh = plsc.VectorSubcoreMesh(core_axis_name="c", subcore_axis_name="s")
    k = functools.partial(
        pl.kernel,
        mesh=mesh,
        out_type=jax.ShapeDtypeStruct(embeddings.shape, embeddings.dtype),
        scratch_types=[pltpu.VMEM((_CHUNK, _DIM), embeddings.dtype)],
    )(_sc_copy)
    return k(embeddings)
